# Initial kernel scaffold; baseline (speedup 1.0000x reference)
import jax
import jax.numpy as jnp
from jax.experimental import pallas as pl


def kernel(x, W_cls, b_cls):
    raise NotImplementedError("write your pallas kernel here")


# speedup vs baseline: 6.3928x; 6.3928x over previous
import functools

import jax
import jax.numpy as jnp
import numpy as np
from jax import lax
from jax.experimental import pallas as pl
from jax.experimental.pallas import tpu as pltpu
from jax.experimental.pallas import tpu_sc as plsc

BALL_BBOX_SIZE = 20.0
_H = 512
_W = 512
_h = 128
_w = 128
_NEG = float("-inf")


def _build_M(W_cls):
    Wf = W_cls[:, :, 0, 0].reshape(2, 3, 4, 4)
    dy = np.arange(4)[:, None, None]
    jj = np.arange(_w)[None, :, None]
    dx = np.arange(4)[None, None, :]
    k_idx = (4 * jj + dx + 0 * dy).reshape(-1)
    m_idx = (dy * _w + jj + 0 * dx).reshape(-1)
    vals = Wf[:, :, (dy + 0 * jj + 0 * dx).reshape(-1), (dx + 0 * jj + 0 * dy).reshape(-1)]
    M = jnp.zeros((2, 3, _W, 4 * _w), jnp.float32)
    return M.at[:, :, k_idx, m_idx].set(vals)


def _build_S():
    S = np.zeros((4, _h, _H), np.float32)
    i = np.arange(_h)
    for dy in range(4):
        S[dy, i, 4 * i + dy] = 1.0
    return jnp.asarray(S)


def _conf_kernel(x_ref, m_ref, s_ref, b_ref, out_ref):
    xb = x_ref[0]
    a = []
    for o in range(2):
        y = None
        for c in range(3):
            t = jax.lax.dot_general(
                xb[c], m_ref[o, c],
                dimension_numbers=(((1,), (0,)), ((), ())),
            )
            y = t if y is None else y + t
        acc = None
        for dy in range(4):
            sel = jax.lax.dot_general(
                s_ref[dy], y[:, dy * _w:(dy + 1) * _w],
                dimension_numbers=(((1,), (0,)), ((), ())),
                precision=jax.lax.Precision.HIGHEST,
            )
            acc = sel if acc is None else acc + sel
        a.append(acc + b_ref[o])
    a0, a1 = a
    m = jnp.maximum(a0, a1)
    e0 = jnp.exp(a0 - m)
    e1 = jnp.exp(a1 - m)
    conf = e1 / (e0 + e1)
    ninf_row = jnp.full((1, _w), _NEG, jnp.float32)
    up = jnp.concatenate([conf[1:], ninf_row], axis=0)
    dn = jnp.concatenate([ninf_row, conf[:-1]], axis=0)
    vm = jnp.maximum(jnp.maximum(conf, up), dn)
    ninf_col = jnp.full((_h, 1), _NEG, jnp.float32)
    lf = jnp.concatenate([vm[:, 1:], ninf_col], axis=1)
    rt = jnp.concatenate([ninf_col, vm[:, :-1]], axis=1)
    pooled = jnp.maximum(jnp.maximum(vm, lf), rt)
    out_ref[0] = jnp.where(conf == pooled, conf, jnp.zeros_like(conf))


def _conf_nms(x, M, S, b_cls):
    B = x.shape[0]
    return pl.pallas_call(
        _conf_kernel,
        grid=(B,),
        in_specs=[
            pl.BlockSpec((1, 3, _H, _W), lambda b: (b, 0, 0, 0)),
            pl.BlockSpec((2, 3, _W, 4 * _w), lambda b: (0, 0, 0, 0)),
            pl.BlockSpec((4, _h, _H), lambda b: (0, 0, 0)),
            pl.BlockSpec(memory_space=pltpu.SMEM),
        ],
        out_specs=pl.BlockSpec((1, _h, _w), lambda b: (b, 0, 0)),
        out_shape=jax.ShapeDtypeStruct((B, _h, _w), jnp.float32),
    )(x, M, S, b_cls)


_N = _h * _w
_NV = _N // 16
_K = 100
_SELA = 128
_SELB = 112
_SEL = _SELA + _SELB
_RES = 112


def _topk_body(conf_hbm, out_hbm, vals, hist, selv, seli, resv, resi, outT, sem):
    del sem
    nc = 2
    wid = lax.axis_index("s") * nc + lax.axis_index("c")
    B = out_hbm.shape[0]

    @pl.when(wid < B)
    def _():
        b = wid
        pltpu.sync_copy(conf_hbm.at[b], vals)

        lanes = lax.iota(jnp.int32, 16)
        ones = jnp.ones((16,), jnp.int32)
        zeros16 = jnp.zeros((16,), jnp.int32)

        def zero_hist(d, _):
            hist[d] = zeros16
            return 0
        lax.fori_loop(0, 256, zero_hist, 0)

        prefix = jnp.int32(0)
        kk = jnp.int32(_K)
        for level, shift in enumerate((24, 16, 8, 0)):
            pfx_hi = prefix >> jnp.int32(shift + 8) if level else jnp.int32(0)

            def data_pass(i, _, shift=shift, level=level, pfx_hi=pfx_hi):
                v = vals[pl.ds(i * 16, 16)]
                u = lax.bitcast_convert_type(v, jnp.int32)
                d = (u >> jnp.int32(shift)) & jnp.int32(0xFF)
                if level == 0:
                    ok = jnp.ones((16,), jnp.bool_)
                else:
                    ok = (u >> jnp.int32(shift + 8)) == pfx_hi
                plsc.addupdate_scatter(hist, (d, lanes), ones, mask=ok)
                return 0
            lax.fori_loop(0, _NV, data_pass, 0)

            def walk(t, carry):
                cum, Bd, nhi = carry
                d = jnp.int32(255) - t
                row = hist[d]
                hist[d] = zeros16
                cnt = jnp.sum(row)
                hit = jnp.logical_and(cum < kk, cum + cnt >= kk)
                Bd = jnp.where(hit, d, Bd)
                nhi = jnp.where(hit, cum, nhi)
                return cum + cnt, Bd, nhi
            _, Bd, nhi = lax.fori_loop(
                0, 256, walk, (jnp.int32(0), jnp.int32(0), jnp.int32(0)))
            prefix = prefix | (Bd << jnp.int32(shift))
            kk = kk - nhi

        T_bits = prefix

        for j in range(_SEL // 16):
            selv[pl.ds(j * 16, 16)] = jnp.full((16,), -1.0, jnp.float32)
            seli[pl.ds(j * 16, 16)] = zeros16
        for j in range(_RES // 16):
            resv[pl.ds(j * 16, 16)] = jnp.zeros((16,), jnp.float32)
            resi[pl.ds(j * 16, 16)] = zeros16

        def collect(i, carry):
            off_gt, off_eq = carry
            v = vals[pl.ds(i * 16, 16)]
            u = lax.bitcast_convert_type(v, jnp.int32)
            idxv = i * 16 + lanes
            m_gt = u > T_bits
            m_eq = u == T_bits
            c_gt = lax.cumsum(m_gt.astype(jnp.int32))
            c_eq = lax.cumsum(m_eq.astype(jnp.int32))
            pos_gt = off_gt + c_gt - 1
            pos_eq = _SELA + off_eq + c_eq - 1
            m_eq = jnp.logical_and(m_eq, pos_eq < _SEL)
            plsc.store_scatter(selv, (pos_gt,), v, mask=m_gt)
            plsc.store_scatter(seli, (pos_gt,), idxv, mask=m_gt)
            plsc.store_scatter(selv, (pos_eq,), v, mask=m_eq)
            plsc.store_scatter(seli, (pos_eq,), idxv, mask=m_eq)
            return off_gt + jnp.sum(m_gt.astype(jnp.int32)), \
                   off_eq + jnp.sum(m_eq.astype(jnp.int32))
        lax.fori_loop(0, _NV, collect, (jnp.int32(0), jnp.int32(0)))

        nsel = _SEL // 16

        def extract(k, _):
            vecs = [selv[pl.ds(j * 16, 16)] for j in range(nsel)]
            m = vecs[0]
            for j in range(1, nsel):
                m = jnp.maximum(m, vecs[j])
            M = jnp.max(m)
            pos = jnp.int32(32767)
            for j in range(nsel):
                slot = j * 16 + lanes
                cand = jnp.min(jnp.where(vecs[j] == M, slot, jnp.int32(32767)))
                pos = jnp.minimum(pos, cand)
            posv = jnp.full((16,), 0, jnp.int32) + pos
            idx = jnp.max(plsc.load_gather(seli, (posv,)))
            lane0 = lanes == 0
            kv = jnp.full((16,), 0, jnp.int32) + k
            plsc.store_scatter(resv, (kv,), jnp.full((16,), 0.0, jnp.float32) + M,
                               mask=lane0)
            plsc.store_scatter(resi, (kv,), jnp.full((16,), 0, jnp.int32) + idx,
                               mask=lane0)
            plsc.store_scatter(selv, (posv,), jnp.full((16,), -2.0, jnp.float32),
                               mask=lane0)
            return 0
        lax.fori_loop(0, _K, extract, 0)

        for j in range(_RES // 16):
            v = resv[pl.ds(j * 16, 16)]
            idx = resi[pl.ds(j * 16, 16)]
            xcol = (idx & jnp.int32(127)).astype(jnp.float32)
            yrow = (idx >> jnp.int32(7)).astype(jnp.float32)
            outT[0, pl.ds(j * 16, 16)] = 4.0 * xcol - 8.5
            outT[1, pl.ds(j * 16, 16)] = 4.0 * yrow - 8.5
            outT[2, pl.ds(j * 16, 16)] = 4.0 * xcol + 11.5
            outT[3, pl.ds(j * 16, 16)] = 4.0 * yrow + 11.5
            outT[4, pl.ds(j * 16, 16)] = v
            outT[5, pl.ds(j * 16, 16)] = jnp.zeros((16,), jnp.float32)
            outT[6, pl.ds(j * 16, 16)] = jnp.zeros((16,), jnp.float32)
            outT[7, pl.ds(j * 16, 16)] = jnp.zeros((16,), jnp.float32)

        pltpu.sync_copy(outT, out_hbm.at[b])


def _topk_decode(conf_flat):
    B = conf_flat.shape[0]
    f = pl.kernel(
        _topk_body,
        out_type=jax.ShapeDtypeStruct((B, 8, _RES), jnp.float32),
        mesh=plsc.VectorSubcoreMesh(core_axis_name="c", subcore_axis_name="s"),
        scratch_types=[
            pltpu.VMEM((_N,), jnp.float32),
            pltpu.VMEM((256, 16), jnp.int32),
            pltpu.VMEM((_SEL,), jnp.float32),
            pltpu.VMEM((_SEL,), jnp.int32),
            pltpu.VMEM((_RES,), jnp.float32),
            pltpu.VMEM((_RES,), jnp.int32),
            pltpu.VMEM((8, _RES), jnp.float32),
            pltpu.SemaphoreType.DMA,
        ],
        compiler_params=pltpu.CompilerParams(needs_layout_passes=False),
    )
    out = f(conf_flat)
    return jnp.transpose(out, (0, 2, 1))[:, :_K, :5]


def kernel(x, W_cls, b_cls):
    M = _build_M(W_cls)
    S = _build_S()
    conf = _conf_nms(x, M, S, b_cls)
    B = x.shape[0]
    return _topk_decode(conf.reshape(B, -1))


# speedup vs baseline: 41.1052x; 6.4299x over previous
import functools

import jax
import jax.numpy as jnp
import numpy as np
from jax import lax
from jax.experimental import pallas as pl
from jax.experimental.pallas import tpu as pltpu
from jax.experimental.pallas import tpu_sc as plsc

BALL_BBOX_SIZE = 20.0
_H = 512
_W = 512
_h = 128
_w = 128
_NEG = float("-inf")


_EB6 = np.broadcast_to(
    np.eye(_w, dtype=np.float32)[:, None, None, :], (_w, 4, 4, _w)).copy()


def _build_M(W_cls):
    Wf = W_cls[:, :, 0, 0].reshape(2, 3, 4, 4)
    Wt = jnp.transpose(Wf, (0, 1, 3, 2))
    M6 = Wt[:, :, None, :, :, None] * jnp.asarray(_EB6)[None, None]
    return M6.reshape(2, 3, _W, 4 * _w)


def _conf_kernel(x_ref, m_ref, b_ref, out_ref):
    xb = x_ref[0]
    a = []
    for o in range(2):
        y = None
        for c in range(3):
            t = jax.lax.dot_general(
                xb[c], m_ref[o, c],
                dimension_numbers=(((1,), (0,)), ((), ())),
            )
            y = t if y is None else y + t
        yr = y.reshape(_h, 4, _H)
        acc = None
        for dy in range(4):
            sel = yr[:, dy, dy * _w:(dy + 1) * _w]
            acc = sel if acc is None else acc + sel
        a.append(acc + b_ref[o])
    a0, a1 = a
    m = jnp.maximum(a0, a1)
    e0 = jnp.exp(a0 - m)
    e1 = jnp.exp(a1 - m)
    conf = e1 / (e0 + e1)
    ninf_row = jnp.full((1, _w), _NEG, jnp.float32)
    up = jnp.concatenate([conf[1:], ninf_row], axis=0)
    dn = jnp.concatenate([ninf_row, conf[:-1]], axis=0)
    vm = jnp.maximum(jnp.maximum(conf, up), dn)
    ninf_col = jnp.full((_h, 1), _NEG, jnp.float32)
    lf = jnp.concatenate([vm[:, 1:], ninf_col], axis=1)
    rt = jnp.concatenate([ninf_col, vm[:, :-1]], axis=1)
    pooled = jnp.maximum(jnp.maximum(vm, lf), rt)
    out_ref[0] = jnp.where(conf == pooled, conf, jnp.zeros_like(conf))


def _conf_nms(x, M, b_cls):
    B = x.shape[0]
    return pl.pallas_call(
        _conf_kernel,
        grid=(B,),
        in_specs=[
            pl.BlockSpec((1, 3, _H, _W), lambda b: (b, 0, 0, 0)),
            pl.BlockSpec((2, 3, _W, 4 * _w), lambda b: (0, 0, 0, 0)),
            pl.BlockSpec(memory_space=pltpu.SMEM),
        ],
        out_specs=pl.BlockSpec((1, _h, _w), lambda b: (b, 0, 0)),
        out_shape=jax.ShapeDtypeStruct((B, _h, _w), jnp.float32),
    )(x, M, b_cls)


_N = _h * _w
_NV = _N // 16
_K = 100
_SEL = 112
_RES = 112


def _topk_body(conf_hbm, out_hbm, vals, Lv, Li, histT, selv, seli, resv, resi,
               outT, sem):
    del sem
    nc = 2
    wid = lax.axis_index("s") * nc + lax.axis_index("c")
    B = out_hbm.shape[0]

    @pl.when(wid < B)
    def _():
        b = wid
        pltpu.sync_copy(conf_hbm.at[b], vals)

        lanes = lax.iota(jnp.int32, 16)
        ones = jnp.ones((16,), jnp.int32)
        zeros16 = jnp.zeros((16,), jnp.int32)

        def zero_hist():
            for d in range(64):
                histT[d] = zeros16
        zero_hist()

        def hist0(i, _):
            for u4 in range(4):
                v = vals[pl.ds(i * 64 + u4 * 16, 16)]
                u = lax.bitcast_convert_type(v, jnp.int32)
                d = u >> jnp.int32(24)
                plsc.addupdate_scatter(histT, (d, lanes), ones)
            return 0
        lax.fori_loop(0, _NV // 4, hist0, 0)

        def walk(kk):
            tots = []
            for c4 in range(4):
                rows = 16 * c4 + lanes
                acc = None
                for l in range(16):
                    g = plsc.load_gather(histT, (rows, zeros16 + l))
                    acc = g if acc is None else acc + g
                tots.append(acc)
            zero_hist()
            rev = [lax.rev(tc, (0,)) for tc in tots]
            cs = [lax.cumsum(r) for r in rev]
            chunk_tot = [jnp.max(c) for c in cs]
            Bd = jnp.int32(-1)
            nhi = jnp.int32(0)
            hi = jnp.int32(0)
            for c4 in (3, 2, 1, 0):
                S = cs[c4] + hi
                mask = S >= kk
                lstar = jnp.min(jnp.where(mask, lanes, jnp.int32(99)))
                sel = lanes == lstar
                S_hit = jnp.max(jnp.where(sel, S, jnp.int32(0)))
                cnt_hit = jnp.max(jnp.where(sel, rev[c4], jnp.int32(0)))
                dcand = jnp.int32(16 * c4 + 15) - lstar
                found = jnp.logical_and(lstar < 16, Bd < 0)
                Bd = jnp.where(found, dcand, Bd)
                nhi = jnp.where(found, S_hit - cnt_hit, nhi)
                hi = hi + chunk_tot[c4]
            return Bd, nhi

        kk = jnp.int32(_K)
        B0, nhi = walk(kk)
        kk = kk - nhi

        def compact(i, off):
            us, ms, idxs = [], [], []
            for g in range(4):
                v = vals[pl.ds(i * 64 + g * 16, 16)]
                u = lax.bitcast_convert_type(v, jnp.int32)
                us.append(u)
                ms.append((u >> jnp.int32(24)) >= B0)
                idxs.append(i * 64 + g * 16 + lanes)
            s = [jnp.sum(m.astype(jnp.int32)) for m in ms]
            offs = [off, off + s[0], off + s[0] + s[1], off + s[0] + s[1] + s[2]]
            for g in range(4):
                plsc.store_compressed(Lv.at[pl.ds(offs[g], 16)], us[g],
                                      mask=ms[g])
                plsc.store_compressed(Li.at[pl.ds(offs[g], 16)], idxs[g],
                                      mask=ms[g])
            return offs[3] + s[3]
        nL = lax.fori_loop(0, _NV // 4, compact, jnp.int32(0))
        nit = (nL + 15) >> jnp.int32(4)

        prefix = B0 << jnp.int32(24)
        for shift in (18, 12, 6, 0):
            pfx_hi = prefix >> jnp.int32(shift + 6)

            def data_pass(i, _, shift=shift, pfx_hi=pfx_hi):
                u = Lv[pl.ds(i * 16, 16)]
                valid = (i * 16 + lanes) < nL
                d = (u >> jnp.int32(shift)) & jnp.int32(63)
                ok = jnp.logical_and(valid,
                                     (u >> jnp.int32(shift + 6)) == pfx_hi)
                plsc.addupdate_scatter(histT, (d, lanes), ones, mask=ok)
                return 0
            lax.fori_loop(0, nit, data_pass, 0)

            Bd, nhi = walk(kk)
            prefix = prefix | (Bd << jnp.int32(shift))
            kk = kk - nhi

        T_bits = prefix

        for j in range(_SEL // 16):
            selv[pl.ds(j * 16, 16)] = jnp.full((16,), -1, jnp.int32)
            seli[pl.ds(j * 16, 16)] = zeros16
        for j in range(_RES // 16):
            resv[pl.ds(j * 16, 16)] = zeros16
            resi[pl.ds(j * 16, 16)] = zeros16

        def coll_gt(i, off):
            u = Lv[pl.ds(i * 16, 16)]
            idx = Li[pl.ds(i * 16, 16)]
            valid = (i * 16 + lanes) < nL
            m = jnp.logical_and(valid, u > T_bits)
            plsc.store_compressed(selv.at[pl.ds(off, 16)], u, mask=m)
            plsc.store_compressed(seli.at[pl.ds(off, 16)], idx, mask=m)
            return off + jnp.sum(m.astype(jnp.int32))
        n_gt = lax.fori_loop(0, nit, coll_gt, jnp.int32(0))

        def coll_eq(i, carry):
            off, left = carry
            u = Lv[pl.ds(i * 16, 16)]
            idx = Li[pl.ds(i * 16, 16)]
            valid = (i * 16 + lanes) < nL
            m = jnp.logical_and(valid, u == T_bits)
            m = jnp.logical_and(m, lax.cumsum(m.astype(jnp.int32)) <= left)
            plsc.store_compressed(selv.at[pl.ds(off, 16)], u, mask=m)
            plsc.store_compressed(seli.at[pl.ds(off, 16)], idx, mask=m)
            take = jnp.sum(m.astype(jnp.int32))
            return off + take, left - take
        lax.fori_loop(0, nit, coll_eq, (n_gt, kk))

        nsel = _SEL // 16

        def extract(k, _):
            vecs = [selv[pl.ds(j * 16, 16)] for j in range(nsel)]
            m = vecs[0]
            for j in range(1, nsel):
                m = jnp.maximum(m, vecs[j])
            Mb = jnp.max(m)
            pv = jnp.where(vecs[0] == Mb, lanes, jnp.int32(32767))
            for j in range(1, nsel):
                slot = j * 16 + lanes
                pv = jnp.minimum(pv, jnp.where(vecs[j] == Mb, slot,
                                               jnp.int32(32767)))
            pos = jnp.min(pv)
            posv = jnp.full((16,), 0, jnp.int32) + pos
            idxg = plsc.load_gather(seli, (posv,))
            lane0 = lanes == 0
            kv = jnp.full((16,), 0, jnp.int32) + k
            plsc.store_scatter(resv, (kv,), jnp.full((16,), 0, jnp.int32) + Mb,
                               mask=lane0)
            plsc.store_scatter(resi, (kv,), idxg, mask=lane0)
            plsc.store_scatter(selv, (posv,), jnp.full((16,), -1, jnp.int32),
                               mask=lane0)
            return 0
        lax.fori_loop(0, _K, extract, 0)

        for j in range(_RES // 16):
            v = lax.bitcast_convert_type(resv[pl.ds(j * 16, 16)], jnp.float32)
            idx = resi[pl.ds(j * 16, 16)]
            xcol = (idx & jnp.int32(127)).astype(jnp.float32)
            yrow = (idx >> jnp.int32(7)).astype(jnp.float32)
            outT[0, pl.ds(j * 16, 16)] = 4.0 * xcol - 8.5
            outT[1, pl.ds(j * 16, 16)] = 4.0 * yrow - 8.5
            outT[2, pl.ds(j * 16, 16)] = 4.0 * xcol + 11.5
            outT[3, pl.ds(j * 16, 16)] = 4.0 * yrow + 11.5
            outT[4, pl.ds(j * 16, 16)] = v
            outT[5, pl.ds(j * 16, 16)] = jnp.zeros((16,), jnp.float32)
            outT[6, pl.ds(j * 16, 16)] = jnp.zeros((16,), jnp.float32)
            outT[7, pl.ds(j * 16, 16)] = jnp.zeros((16,), jnp.float32)

        pltpu.sync_copy(outT, out_hbm.at[b])


def _topk_decode(conf_flat):
    B = conf_flat.shape[0]
    f = pl.kernel(
        _topk_body,
        out_type=jax.ShapeDtypeStruct((B, 8, _RES), jnp.float32),
        mesh=plsc.VectorSubcoreMesh(core_axis_name="c", subcore_axis_name="s"),
        scratch_types=[
            pltpu.VMEM((_N,), jnp.float32),
            pltpu.VMEM((_N,), jnp.int32),
            pltpu.VMEM((_N,), jnp.int32),
            pltpu.VMEM((64, 16), jnp.int32),
            pltpu.VMEM((_SEL,), jnp.int32),
            pltpu.VMEM((_SEL,), jnp.int32),
            pltpu.VMEM((_RES,), jnp.int32),
            pltpu.VMEM((_RES,), jnp.int32),
            pltpu.VMEM((8, _RES), jnp.float32),
            pltpu.SemaphoreType.DMA,
        ],
        compiler_params=pltpu.CompilerParams(needs_layout_passes=False),
    )
    out = f(conf_flat)
    return jnp.transpose(out, (0, 2, 1))[:, :_K, :5]


def kernel(x, W_cls, b_cls):
    M = _build_M(W_cls)
    conf = _conf_nms(x, M, b_cls)
    B = x.shape[0]
    return _topk_decode(conf.reshape(B, -1))


# speedup vs baseline: 48.7730x; 1.1865x over previous
import functools

import jax
import jax.numpy as jnp
import numpy as np
from jax import lax
from jax.experimental import pallas as pl
from jax.experimental.pallas import tpu as pltpu
from jax.experimental.pallas import tpu_sc as plsc

BALL_BBOX_SIZE = 20.0
_H = 512
_W = 512
_h = 128
_w = 128
_NEG = float("-inf")


_EB2 = np.broadcast_to(
    np.eye(_w, dtype=np.float32)[:, None, :], (_w, 4, _w)).copy()


def _build_M(W_cls):
    Wf = W_cls[:, :, 0, 0].reshape(2, 3, 4, 4)
    M6 = Wf[:, :, :, None, :, None] * jnp.asarray(_EB2)[None, None, None]
    return M6.reshape(2, 3, 4, _W, _w)


def _conf_kernel(x_ref, m_ref, b_ref, out_ref):
    xb = x_ref[0]
    xs = []
    for c in range(3):
        xr = xb[c].reshape(_h, 4, _H)
        xs.append([xr[:, dy, :] for dy in range(4)])
    a = []
    for o in range(2):
        acc = None
        for dy in range(4):
            y = None
            for c in range(3):
                t = jax.lax.dot_general(
                    xs[c][dy], m_ref[o, c, dy],
                    dimension_numbers=(((1,), (0,)), ((), ())),
                )
                y = t if y is None else y + t
            acc = y if acc is None else acc + y
        a.append(acc + b_ref[o])
    a0, a1 = a
    m = jnp.maximum(a0, a1)
    e0 = jnp.exp(a0 - m)
    e1 = jnp.exp(a1 - m)
    conf = e1 / (e0 + e1)
    ninf_row = jnp.full((1, _w), _NEG, jnp.float32)
    up = jnp.concatenate([conf[1:], ninf_row], axis=0)
    dn = jnp.concatenate([ninf_row, conf[:-1]], axis=0)
    vm = jnp.maximum(jnp.maximum(conf, up), dn)
    ninf_col = jnp.full((_h, 1), _NEG, jnp.float32)
    lf = jnp.concatenate([vm[:, 1:], ninf_col], axis=1)
    rt = jnp.concatenate([ninf_col, vm[:, :-1]], axis=1)
    pooled = jnp.maximum(jnp.maximum(vm, lf), rt)
    out_ref[0] = jnp.where(conf == pooled, conf, jnp.zeros_like(conf))


def _conf_nms(x, M, b_cls):
    B = x.shape[0]
    return pl.pallas_call(
        _conf_kernel,
        grid=(B,),
        in_specs=[
            pl.BlockSpec((1, 3, _H, _W), lambda b: (b, 0, 0, 0)),
            pl.BlockSpec((2, 3, 4, _W, _w), lambda b: (0, 0, 0, 0, 0)),
            pl.BlockSpec(memory_space=pltpu.SMEM),
        ],
        out_specs=pl.BlockSpec((1, _h, _w), lambda b: (b, 0, 0)),
        out_shape=jax.ShapeDtypeStruct((B, _h, _w), jnp.float32),
    )(x, M, b_cls)


_N = _h * _w
_NV = _N // 16
_K = 100
_SEL = 112
_RES = 112


def _topk_body(conf_hbm, out_hbm, vals, Lv, Li, histT, selv, seli, resv, resi,
               outT, sem):
    del sem
    nc = 2
    wid = lax.axis_index("s") * nc + lax.axis_index("c")
    B = out_hbm.shape[0]

    @pl.when(wid < B)
    def _():
        b = wid
        pltpu.sync_copy(conf_hbm.at[b], vals)

        lanes = lax.iota(jnp.int32, 16)
        ones = jnp.ones((16,), jnp.int32)
        zeros16 = jnp.zeros((16,), jnp.int32)

        def zero_hist():
            for d in range(64):
                histT[d] = zeros16
        zero_hist()

        def hist0(i, _):
            for u4 in range(4):
                v = vals[pl.ds(i * 64 + u4 * 16, 16)]
                u = lax.bitcast_convert_type(v, jnp.int32)
                d = u >> jnp.int32(24)
                plsc.addupdate_scatter(histT, (d, lanes), ones)
            return 0
        lax.fori_loop(0, _NV // 4, hist0, 0)

        def walk(kk):
            tots = []
            for c4 in range(4):
                rows = 16 * c4 + lanes
                acc = None
                for l in range(16):
                    g = plsc.load_gather(histT, (rows, zeros16 + l))
                    acc = g if acc is None else acc + g
                tots.append(acc)
            zero_hist()
            rev = [lax.rev(tc, (0,)) for tc in tots]
            cs = [lax.cumsum(r) for r in rev]
            chunk_tot = [jnp.max(c) for c in cs]
            Bd = jnp.int32(-1)
            nhi = jnp.int32(0)
            hi = jnp.int32(0)
            for c4 in (3, 2, 1, 0):
                S = cs[c4] + hi
                mask = S >= kk
                lstar = jnp.min(jnp.where(mask, lanes, jnp.int32(99)))
                sel = lanes == lstar
                S_hit = jnp.max(jnp.where(sel, S, jnp.int32(0)))
                cnt_hit = jnp.max(jnp.where(sel, rev[c4], jnp.int32(0)))
                dcand = jnp.int32(16 * c4 + 15) - lstar
                found = jnp.logical_and(lstar < 16, Bd < 0)
                Bd = jnp.where(found, dcand, Bd)
                nhi = jnp.where(found, S_hit - cnt_hit, nhi)
                hi = hi + chunk_tot[c4]
            return Bd, nhi

        kk = jnp.int32(_K)
        B0, nhi = walk(kk)
        kk = kk - nhi

        def compact(i, off):
            us, ms, idxs = [], [], []
            for g in range(4):
                v = vals[pl.ds(i * 64 + g * 16, 16)]
                u = lax.bitcast_convert_type(v, jnp.int32)
                us.append(u)
                ms.append((u >> jnp.int32(24)) >= B0)
                idxs.append(i * 64 + g * 16 + lanes)
            s = [jnp.sum(m.astype(jnp.int32)) for m in ms]
            offs = [off, off + s[0], off + s[0] + s[1], off + s[0] + s[1] + s[2]]
            for g in range(4):
                plsc.store_compressed(Lv.at[pl.ds(offs[g], 16)], us[g],
                                      mask=ms[g])
                plsc.store_compressed(Li.at[pl.ds(offs[g], 16)], idxs[g],
                                      mask=ms[g])
            return offs[3] + s[3]
        nL = lax.fori_loop(0, _NV // 4, compact, jnp.int32(0))
        nit = (nL + 15) >> jnp.int32(4)

        prefix = B0 << jnp.int32(24)
        for shift in (18, 12, 6, 0):
            pfx_hi = prefix >> jnp.int32(shift + 6)

            def data_pass(i, _, shift=shift, pfx_hi=pfx_hi):
                u = Lv[pl.ds(i * 16, 16)]
                valid = (i * 16 + lanes) < nL
                d = (u >> jnp.int32(shift)) & jnp.int32(63)
                ok = jnp.logical_and(valid,
                                     (u >> jnp.int32(shift + 6)) == pfx_hi)
                plsc.addupdate_scatter(histT, (d, lanes), ones, mask=ok)
                return 0
            lax.fori_loop(0, nit, data_pass, 0)

            Bd, nhi = walk(kk)
            prefix = prefix | (Bd << jnp.int32(shift))
            kk = kk - nhi

        T_bits = prefix

        for j in range(_SEL // 16):
            selv[pl.ds(j * 16, 16)] = jnp.full((16,), -1, jnp.int32)
            seli[pl.ds(j * 16, 16)] = zeros16
        for j in range(_RES // 16):
            resv[pl.ds(j * 16, 16)] = zeros16
            resi[pl.ds(j * 16, 16)] = zeros16

        def coll_gt(i, off):
            u = Lv[pl.ds(i * 16, 16)]
            idx = Li[pl.ds(i * 16, 16)]
            valid = (i * 16 + lanes) < nL
            m = jnp.logical_and(valid, u > T_bits)
            plsc.store_compressed(selv.at[pl.ds(off, 16)], u, mask=m)
            plsc.store_compressed(seli.at[pl.ds(off, 16)], idx, mask=m)
            return off + jnp.sum(m.astype(jnp.int32))
        n_gt = lax.fori_loop(0, nit, coll_gt, jnp.int32(0))

        def coll_eq(i, carry):
            off, left = carry
            u = Lv[pl.ds(i * 16, 16)]
            idx = Li[pl.ds(i * 16, 16)]
            valid = (i * 16 + lanes) < nL
            m = jnp.logical_and(valid, u == T_bits)
            m = jnp.logical_and(m, lax.cumsum(m.astype(jnp.int32)) <= left)
            plsc.store_compressed(selv.at[pl.ds(off, 16)], u, mask=m)
            plsc.store_compressed(seli.at[pl.ds(off, 16)], idx, mask=m)
            take = jnp.sum(m.astype(jnp.int32))
            return off + take, left - take
        lax.fori_loop(0, nit, coll_eq, (n_gt, kk))

        nsel = _SEL // 16

        def extract(k, _):
            vecs = [selv[pl.ds(j * 16, 16)] for j in range(nsel)]
            m = vecs[0]
            for j in range(1, nsel):
                m = jnp.maximum(m, vecs[j])
            Mb = jnp.max(m)
            pv = jnp.where(vecs[0] == Mb, lanes, jnp.int32(32767))
            for j in range(1, nsel):
                slot = j * 16 + lanes
                pv = jnp.minimum(pv, jnp.where(vecs[j] == Mb, slot,
                                               jnp.int32(32767)))
            pos = jnp.min(pv)
            posv = jnp.full((16,), 0, jnp.int32) + pos
            idxg = plsc.load_gather(seli, (posv,))
            lane0 = lanes == 0
            kv = jnp.full((16,), 0, jnp.int32) + k
            plsc.store_scatter(resv, (kv,), jnp.full((16,), 0, jnp.int32) + Mb,
                               mask=lane0)
            plsc.store_scatter(resi, (kv,), idxg, mask=lane0)
            plsc.store_scatter(selv, (posv,), jnp.full((16,), -1, jnp.int32),
                               mask=lane0)
            return 0
        lax.fori_loop(0, _K, extract, 0)

        for j in range(_RES // 16):
            v = lax.bitcast_convert_type(resv[pl.ds(j * 16, 16)], jnp.float32)
            idx = resi[pl.ds(j * 16, 16)]
            xcol = (idx & jnp.int32(127)).astype(jnp.float32)
            yrow = (idx >> jnp.int32(7)).astype(jnp.float32)
            outT[0, pl.ds(j * 16, 16)] = 4.0 * xcol - 8.5
            outT[1, pl.ds(j * 16, 16)] = 4.0 * yrow - 8.5
            outT[2, pl.ds(j * 16, 16)] = 4.0 * xcol + 11.5
            outT[3, pl.ds(j * 16, 16)] = 4.0 * yrow + 11.5
            outT[4, pl.ds(j * 16, 16)] = v
            outT[5, pl.ds(j * 16, 16)] = jnp.zeros((16,), jnp.float32)
            outT[6, pl.ds(j * 16, 16)] = jnp.zeros((16,), jnp.float32)
            outT[7, pl.ds(j * 16, 16)] = jnp.zeros((16,), jnp.float32)

        pltpu.sync_copy(outT, out_hbm.at[b])


def _topk_decode(conf_flat):
    B = conf_flat.shape[0]
    f = pl.kernel(
        _topk_body,
        out_type=jax.ShapeDtypeStruct((B, 8, _RES), jnp.float32),
        mesh=plsc.VectorSubcoreMesh(core_axis_name="c", subcore_axis_name="s"),
        scratch_types=[
            pltpu.VMEM((_N,), jnp.float32),
            pltpu.VMEM((_N,), jnp.int32),
            pltpu.VMEM((_N,), jnp.int32),
            pltpu.VMEM((64, 16), jnp.int32),
            pltpu.VMEM((_SEL,), jnp.int32),
            pltpu.VMEM((_SEL,), jnp.int32),
            pltpu.VMEM((_RES,), jnp.int32),
            pltpu.VMEM((_RES,), jnp.int32),
            pltpu.VMEM((8, _RES), jnp.float32),
            pltpu.SemaphoreType.DMA,
        ],
        compiler_params=pltpu.CompilerParams(needs_layout_passes=False),
    )
    out = f(conf_flat)
    return jnp.transpose(out, (0, 2, 1))[:, :_K, :5]


def kernel(x, W_cls, b_cls):
    M = _build_M(W_cls)
    conf = _conf_nms(x, M, b_cls)
    B = x.shape[0]
    return _topk_decode(conf.reshape(B, -1))
